# baseline (speedup 1.0000x reference)
import functools

import jax
import jax.numpy as jnp
from jax import lax
from jax.experimental import pallas as pl
from jax.experimental.pallas import tpu as pltpu
from jax.experimental.pallas import tpu_sc as plsc

VOCAB = 30522
HIDDEN = 768
BATCH = 4
SEQ = 2048

NC, NS, L = 2, 16, 16
NW = NC * NS
TOK = BATCH * SEQ
TPW = TOK // NW
CH = 64
NCH = TPW // CH
WPR = SEQ // TPW
HB = HIDDEN // L

_mesh = plsc.VectorSubcoreMesh(core_axis_name="c", subcore_axis_name="s")


@functools.partial(
    pl.kernel,
    mesh=_mesh,
    out_type=[
        jax.ShapeDtypeStruct((TOK, HIDDEN), jnp.float32),
        jax.ShapeDtypeStruct((NW, HIDDEN), jnp.float32),
    ],
    scratch_types=[
        pltpu.VMEM((NCH, CH), jnp.int32),
        pltpu.VMEM((CH, HIDDEN), jnp.float32),
        pltpu.VMEM((L,), jnp.int32),
        pltpu.VMEM((HIDDEN,), jnp.float32),
        pltpu.SemaphoreType.DMA,
    ],
)
def _embed_pool_sc(ids_hbm, vlen_hbm, table_hbm, out_hbm, part_hbm,
                   idx_v, rows_v, vlen_v, acc_v, sem):
    wid = lax.axis_index("s") * NC + lax.axis_index("c")
    base = wid * TPW
    b = wid // WPR
    p0 = (wid % WPR) * TPW

    pltpu.sync_copy(ids_hbm.at[wid], idx_v)
    pltpu.sync_copy(vlen_hbm.at[b], vlen_v)
    vlen = vlen_v[...][0]

    for d in range(HB):
        acc_v[pl.ds(d * L, L)] = jnp.zeros((L,), jnp.float32)

    for c in range(NCH):
        pltpu.async_copy(table_hbm.at[idx_v.at[c]], rows_v, sem).wait()
        pltpu.sync_copy(rows_v, out_hbm.at[pl.ds(base + c * CH, CH)])
        n = jnp.clip(vlen - (p0 + c * CH), 0, CH)

        def _pool(i, _):
            for d in range(HB):
                plsc.addupdate(acc_v.at[pl.ds(d * L, L)],
                               rows_v[i, pl.ds(d * L, L)])
            return 0

        lax.fori_loop(0, n, _pool, 0)

    pltpu.sync_copy(acc_v, part_hbm.at[wid])


def _vlen_body(m_ref, o_ref):
    vl = jnp.sum(m_ref[...], axis=1)
    o_ref[...] = jnp.broadcast_to(vl[:, None], (BATCH, L))


def _combine_body(p_ref, o_ref):
    o_ref[...] = jnp.sum(p_ref[...], axis=1)


def kernel(input_ids, attn_mask, W):
    ids = input_ids.astype(jnp.int32).reshape(NW, NCH, CH)
    mask = attn_mask.astype(jnp.int32).reshape(BATCH, SEQ)
    vlen16 = pl.pallas_call(
        _vlen_body,
        out_shape=jax.ShapeDtypeStruct((BATCH, L), jnp.int32),
    )(mask)
    out_flat, part = _embed_pool_sc(ids, vlen16, W)
    pooled = pl.pallas_call(
        _combine_body,
        out_shape=jax.ShapeDtypeStruct((BATCH, HIDDEN), jnp.float32),
    )(part.reshape(BATCH, NW // BATCH, HIDDEN))
    return (out_flat.reshape(BATCH, SEQ, HIDDEN), pooled)
